# baseline (speedup 1.0000x reference)
import functools

import jax
import jax.numpy as jnp
from jax.experimental import pallas as pl
from jax.experimental.pallas import tpu as pltpu


def _se_body(x_ref, p_ref, o_ref, *, inv_hw, C):
    x = x_ref[...]
    s = jnp.sum(x, axis=2, keepdims=True)
    m = jnp.max(x, axis=2, keepdims=True)
    avg = s * inv_hw

    w1a = p_ref[0:C, :]
    w1m = p_ref[C:2 * C, :]
    b1c = p_ref[2 * C:3 * C, :]
    w2g = p_ref[3 * C:4 * C, :]
    w2b = p_ref[4 * C:5 * C, :]
    b2gc = p_ref[5 * C:6 * C, :]
    b2bc = p_ref[6 * C:7 * C, :]

    t = avg * w1a + m * w1m + b1c
    h = jnp.sum(t, axis=1, keepdims=True)
    h = h * jnp.tanh(jax.nn.softplus(h))

    gam = jnp.sum(w2g * h + b2gc, axis=2, keepdims=True)
    bet = jnp.sum(w2b * h + b2bc, axis=2, keepdims=True)
    scale = jax.nn.sigmoid(gam)

    o_ref[...] = (scale * x + bet).astype(o_ref.dtype)


def kernel(x_nchw, w1, b1, w2, b2):
    B, C, H, W = x_nchw.shape
    HW = H * W
    hidden = w1.shape[0]
    x = x_nchw.reshape(B, C, HW)
    f32 = jnp.float32

    w1a = w1[:, :C].T.astype(f32)
    w1m = w1[:, C:].T.astype(f32)
    b1c = jnp.broadcast_to(b1.astype(f32)[None, :] / C, (C, hidden))
    w2g = w2[:C, :].astype(f32)
    w2b = w2[C:, :].astype(f32)
    b2gc = jnp.broadcast_to(b2[:C, None].astype(f32) / hidden, (C, hidden))
    b2bc = jnp.broadcast_to(b2[C:, None].astype(f32) / hidden, (C, hidden))
    pack = jnp.concatenate([w1a, w1m, b1c, w2g, w2b, b2gc, b2bc], axis=0)

    per_image = C * HW * x.dtype.itemsize
    bt = 1
    for d in range(1, B + 1):
        if B % d == 0 and 4 * d * per_image <= 16 * 2**20 and B // d >= 2:
            bt = d

    body = functools.partial(_se_body, inv_hw=1.0 / HW, C=C)
    out = pl.pallas_call(
        body,
        out_shape=jax.ShapeDtypeStruct((B, C, HW), x.dtype),
        grid=(B // bt,),
        in_specs=[
            pl.BlockSpec((bt, C, HW), lambda i: (i, 0, 0)),
            pl.BlockSpec((7 * C, hidden), lambda i: (0, 0)),
        ],
        out_specs=pl.BlockSpec((bt, C, HW), lambda i: (i, 0, 0)),
        compiler_params=pltpu.CompilerParams(
            dimension_semantics=("arbitrary",),
            vmem_limit_bytes=64 * 2**20,
        ),
    )(x, pack)

    return out.reshape(B, C, H, W)


# speedup vs baseline: 1.0083x; 1.0083x over previous
import functools

import jax
import jax.numpy as jnp
from jax.experimental import pallas as pl
from jax.experimental.pallas import tpu as pltpu


def _se_body(x_ref, p_ref, o_ref, *, inv_hw, C, bt, n_cols):
    HW = x_ref.shape[2]
    lane_w = HW // n_cols
    acc_s = x_ref[:, :, 0:lane_w]
    acc_m = acc_s
    for j in range(1, n_cols):
        c = x_ref[:, :, j * lane_w:(j + 1) * lane_w]
        acc_s = acc_s + c
        acc_m = jnp.maximum(acc_m, c)
    s = jnp.sum(acc_s, axis=2, keepdims=True)
    m = jnp.max(acc_m, axis=2, keepdims=True)
    avg = s * inv_hw

    w1a = p_ref[0:C, :]
    w1m = p_ref[C:2 * C, :]
    b1c = p_ref[2 * C:3 * C, :]
    w2g = p_ref[3 * C:4 * C, :]
    w2b = p_ref[4 * C:5 * C, :]
    b2gc = p_ref[5 * C:6 * C, :]
    b2bc = p_ref[6 * C:7 * C, :]

    t = avg * w1a + m * w1m + b1c
    h = jnp.sum(t, axis=1, keepdims=True)
    h = h * jnp.tanh(jax.nn.softplus(h))

    gam = jnp.sum(w2g * h + b2gc, axis=2, keepdims=True)
    bet = jnp.sum(w2b * h + b2bc, axis=2, keepdims=True)
    scale = jax.nn.sigmoid(gam)

    o_ref[...] = (scale * x_ref[...] + bet).astype(o_ref.dtype)


def kernel(x_nchw, w1, b1, w2, b2):
    B, C, H, W = x_nchw.shape
    HW = H * W
    hidden = w1.shape[0]
    x = x_nchw.reshape(B, C, HW)
    f32 = jnp.float32

    w1a = w1[:, :C].T.astype(f32)
    w1m = w1[:, C:].T.astype(f32)
    b1c = jnp.broadcast_to(b1.astype(f32)[None, :] / C, (C, hidden))
    w2g = w2[:C, :].astype(f32)
    w2b = w2[C:, :].astype(f32)
    b2gc = jnp.broadcast_to(b2[:C, None].astype(f32) / hidden, (C, hidden))
    b2bc = jnp.broadcast_to(b2[C:, None].astype(f32) / hidden, (C, hidden))
    pack = jnp.concatenate([w1a, w1m, b1c, w2g, w2b, b2gc, b2bc], axis=0)

    per_image = C * HW * x.dtype.itemsize
    bt = 1
    for d in range(1, B + 1):
        if B % d == 0 and 4 * d * per_image <= 48 * 2**20 and B // d >= 2:
            bt = d

    n_cols = 8 if HW % (8 * 128) == 0 else 1
    body = functools.partial(_se_body, inv_hw=1.0 / HW, C=C, bt=bt,
                             n_cols=n_cols)
    out = pl.pallas_call(
        body,
        out_shape=jax.ShapeDtypeStruct((B, C, HW), x.dtype),
        grid=(B // bt,),
        in_specs=[
            pl.BlockSpec((bt, C, HW), lambda i: (i, 0, 0)),
            pl.BlockSpec((7 * C, hidden), lambda i: (0, 0)),
        ],
        out_specs=pl.BlockSpec((bt, C, HW), lambda i: (i, 0, 0)),
        compiler_params=pltpu.CompilerParams(
            dimension_semantics=("arbitrary",),
            vmem_limit_bytes=64 * 2**20,
        ),
    )(x, pack)

    return out.reshape(B, C, H, W)


# speedup vs baseline: 1.0086x; 1.0003x over previous
import functools

import jax
import jax.numpy as jnp
from jax.experimental import pallas as pl
from jax.experimental.pallas import tpu as pltpu


def _se_block_math(xb, p_ref, *, inv_hw, C):
    s = jnp.sum(xb, axis=2, keepdims=True)
    m = jnp.max(xb, axis=2, keepdims=True)
    avg = s * inv_hw

    w1a = p_ref[0:C, :]
    w1m = p_ref[C:2 * C, :]
    b1c = p_ref[2 * C:3 * C, :]
    w2g = p_ref[3 * C:4 * C, :]
    w2b = p_ref[4 * C:5 * C, :]
    b2gc = p_ref[5 * C:6 * C, :]
    b2bc = p_ref[6 * C:7 * C, :]

    t = avg * w1a + m * w1m + b1c
    h = jnp.sum(t, axis=1, keepdims=True)
    h = h * jnp.tanh(jax.nn.softplus(h))

    gam = jnp.sum(w2g * h + b2gc, axis=2, keepdims=True)
    bet = jnp.sum(w2b * h + b2bc, axis=2, keepdims=True)
    scale = jax.nn.sigmoid(gam)
    return scale * xb + bet


def _se_pipe(x_hbm, p_ref, o_hbm, x_buf, o_buf, in_sem, out_sem,
             *, inv_hw, C, bt, n_steps):
    def dma_in(slot, step):
        pltpu.make_async_copy(x_hbm.at[pl.ds(step * bt, bt)],
                              x_buf.at[slot], in_sem.at[slot]).start()

    def wait_in(slot):
        pltpu.make_async_copy(x_hbm.at[pl.ds(0, bt)],
                              x_buf.at[slot], in_sem.at[slot]).wait()

    def dma_out(slot, step):
        pltpu.make_async_copy(o_buf.at[slot],
                              o_hbm.at[pl.ds(step * bt, bt)],
                              out_sem.at[slot]).start()

    def wait_out(slot):
        pltpu.make_async_copy(o_buf.at[slot],
                              o_hbm.at[pl.ds(0, bt)],
                              out_sem.at[slot]).wait()

    dma_in(0, 0)
    if n_steps > 1:
        dma_in(1, 1)
    for k in range(n_steps):
        cur = k % 2
        wait_in(cur)
        if k >= 2:
            wait_out(cur)
        y = _se_block_math(x_buf[cur], p_ref, inv_hw=inv_hw, C=C)
        o_buf[cur] = y.astype(o_buf.dtype)
        dma_out(cur, k)
        if k + 2 < n_steps:
            dma_in(cur, k + 2)
    if n_steps >= 2:
        wait_out((n_steps - 2) % 2)
    wait_out((n_steps - 1) % 2)


def kernel(x_nchw, w1, b1, w2, b2):
    B, C, H, W = x_nchw.shape
    HW = H * W
    hidden = w1.shape[0]
    x = x_nchw.reshape(B, C, HW)
    f32 = jnp.float32

    w1a = w1[:, :C].T.astype(f32)
    w1m = w1[:, C:].T.astype(f32)
    b1c = jnp.broadcast_to(b1.astype(f32)[None, :] / C, (C, hidden))
    w2g = w2[:C, :].astype(f32)
    w2b = w2[C:, :].astype(f32)
    b2gc = jnp.broadcast_to(b2[:C, None].astype(f32) / hidden, (C, hidden))
    b2bc = jnp.broadcast_to(b2[C:, None].astype(f32) / hidden, (C, hidden))
    pack = jnp.concatenate([w1a, w1m, b1c, w2g, w2b, b2gc, b2bc], axis=0)

    per_image = C * HW * x.dtype.itemsize
    bt = 1
    for d in range(1, B + 1):
        if B % d == 0 and 4 * d * per_image <= 48 * 2**20 and B // d >= 2:
            bt = d
    n_steps = B // bt

    body = functools.partial(_se_pipe, inv_hw=1.0 / HW, C=C, bt=bt,
                             n_steps=n_steps)
    out = pl.pallas_call(
        body,
        out_shape=jax.ShapeDtypeStruct((B, C, HW), x.dtype),
        in_specs=[
            pl.BlockSpec(memory_space=pl.ANY),
            pl.BlockSpec(memory_space=pltpu.MemorySpace.VMEM),
        ],
        out_specs=pl.BlockSpec(memory_space=pl.ANY),
        scratch_shapes=[
            pltpu.VMEM((2, bt, C, HW), f32),
            pltpu.VMEM((2, bt, C, HW), f32),
            pltpu.SemaphoreType.DMA((2,)),
            pltpu.SemaphoreType.DMA((2,)),
        ],
        compiler_params=pltpu.CompilerParams(
            vmem_limit_bytes=64 * 2**20,
        ),
    )(x, pack)

    return out.reshape(B, C, H, W)


# speedup vs baseline: 1.0226x; 1.0138x over previous
import functools

import jax
import jax.numpy as jnp
from jax.experimental import pallas as pl
from jax.experimental.pallas import tpu as pltpu


def _se_block_math(xb, p_ref, *, inv_hw, C):
    s = jnp.sum(xb, axis=2, keepdims=True)
    m = jnp.max(xb, axis=2, keepdims=True)
    avg = s * inv_hw

    w1a = p_ref[0:C, :]
    w1m = p_ref[C:2 * C, :]
    b1c = p_ref[2 * C:3 * C, :]
    w2g = p_ref[3 * C:4 * C, :]
    w2b = p_ref[4 * C:5 * C, :]
    b2gc = p_ref[5 * C:6 * C, :]
    b2bc = p_ref[6 * C:7 * C, :]

    t = avg * w1a + m * w1m + b1c
    h = jnp.sum(t, axis=1, keepdims=True)
    h = h * jnp.tanh(jax.nn.softplus(h))

    gam = jnp.sum(w2g * h + b2gc, axis=2, keepdims=True)
    bet = jnp.sum(w2b * h + b2bc, axis=2, keepdims=True)
    scale = jax.nn.sigmoid(gam)
    return scale * xb + bet


def _se_pipe(x_hbm, w1_ref, b1_ref, w2_ref, b2_ref, o_hbm,
             p_ref, x_buf, o_buf, in_sem, out_sem,
             *, inv_hw, C, hidden, bt, n_steps):
    p_ref[0:C, :] = w1_ref[:, 0:C].T
    p_ref[C:2 * C, :] = w1_ref[:, C:2 * C].T
    p_ref[2 * C:3 * C, :] = jnp.broadcast_to(
        b1_ref[...] * (1.0 / C), (C, hidden))
    p_ref[3 * C:4 * C, :] = w2_ref[0:C, :]
    p_ref[4 * C:5 * C, :] = w2_ref[C:2 * C, :]
    p_ref[5 * C:6 * C, :] = jnp.broadcast_to(
        b2_ref[0:C, :] * (1.0 / hidden), (C, hidden))
    p_ref[6 * C:7 * C, :] = jnp.broadcast_to(
        b2_ref[C:2 * C, :] * (1.0 / hidden), (C, hidden))
    def dma_in(slot, step):
        pltpu.make_async_copy(x_hbm.at[pl.ds(step * bt, bt)],
                              x_buf.at[slot], in_sem.at[slot]).start()

    def wait_in(slot):
        pltpu.make_async_copy(x_hbm.at[pl.ds(0, bt)],
                              x_buf.at[slot], in_sem.at[slot]).wait()

    def dma_out(slot, step):
        pltpu.make_async_copy(o_buf.at[slot],
                              o_hbm.at[pl.ds(step * bt, bt)],
                              out_sem.at[slot]).start()

    def wait_out(slot):
        pltpu.make_async_copy(o_buf.at[slot],
                              o_hbm.at[pl.ds(0, bt)],
                              out_sem.at[slot]).wait()

    dma_in(0, 0)
    if n_steps > 1:
        dma_in(1, 1)
    for k in range(n_steps):
        cur = k % 2
        wait_in(cur)
        if k >= 2:
            wait_out(cur)
        y = _se_block_math(x_buf[cur], p_ref, inv_hw=inv_hw, C=C)
        o_buf[cur] = y.astype(o_buf.dtype)
        dma_out(cur, k)
        if k + 2 < n_steps:
            dma_in(cur, k + 2)
    if n_steps >= 2:
        wait_out((n_steps - 2) % 2)
    wait_out((n_steps - 1) % 2)


def kernel(x_nchw, w1, b1, w2, b2):
    B, C, H, W = x_nchw.shape
    HW = H * W
    hidden = w1.shape[0]
    x = x_nchw.reshape(B, C, HW)
    f32 = jnp.float32

    w1f = w1.astype(f32)
    b1f = b1.astype(f32).reshape(1, hidden)
    w2f = w2.astype(f32)
    b2f = b2.astype(f32).reshape(2 * C, 1)

    per_image = C * HW * x.dtype.itemsize
    bt = 1
    for d in range(1, B + 1):
        if B % d == 0 and 4 * d * per_image <= 48 * 2**20 and B // d >= 2:
            bt = d
    n_steps = B // bt

    body = functools.partial(_se_pipe, inv_hw=1.0 / HW, C=C,
                             hidden=hidden, bt=bt, n_steps=n_steps)
    vmem = pl.BlockSpec(memory_space=pltpu.MemorySpace.VMEM)
    out = pl.pallas_call(
        body,
        out_shape=jax.ShapeDtypeStruct((B, C, HW), x.dtype),
        in_specs=[pl.BlockSpec(memory_space=pl.ANY),
                  vmem, vmem, vmem, vmem],
        out_specs=pl.BlockSpec(memory_space=pl.ANY),
        scratch_shapes=[
            pltpu.VMEM((7 * C, hidden), f32),
            pltpu.VMEM((2, bt, C, HW), f32),
            pltpu.VMEM((2, bt, C, HW), f32),
            pltpu.SemaphoreType.DMA((2,)),
            pltpu.SemaphoreType.DMA((2,)),
        ],
        compiler_params=pltpu.CompilerParams(
            vmem_limit_bytes=64 * 2**20,
        ),
    )(x, w1f, b1f, w2f, b2f)

    return out.reshape(B, C, H, W)
